# Initial kernel scaffold; baseline (speedup 1.0000x reference)
import jax
import jax.numpy as jnp
from jax.experimental import pallas as pl


def kernel(x, edge_index, W1l, W1r, b1, W2l, W2r, b2, Wt, bt, Wi, bi, Wm, bm):
    raise NotImplementedError("write your pallas kernel here")


# speedup vs baseline: 4.3310x; 4.3310x over previous
import functools

import jax
import jax.numpy as jnp
from jax import lax
from jax.experimental import pallas as pl
from jax.experimental.pallas import tpu as pltpu
from jax.experimental.pallas import tpu_sc as plsc

N = 10000
D = 128
HD = D // 2
R = 10240
NC, NS, L = 2, 16, 16
CH = 128
E = 320000
ECHUNKS = -(-E // (NS * CH))
ET = ECHUNKS * CH
EPAD = ET * NS
RPT = R // NS
CW = 16


def _make_sc_agg(with_counts):
    mesh = plsc.VectorSubcoreMesh(core_axis_name="c", subcore_axis_name="s")

    out_type = [jax.ShapeDtypeStruct((NC, R, HD), jnp.float32)]
    scratch = [
        pltpu.VMEM((CH,), jnp.int32),
        pltpu.VMEM((CH,), jnp.int32),
        pltpu.VMEM((CH, HD), jnp.float32),
        pltpu.VMEM((CH, HD), jnp.float32),
        pltpu.SemaphoreType.DMA,
        pltpu.VMEM_SHARED((R, HD), jnp.float32),
    ]
    if with_counts:
        out_type.append(jax.ShapeDtypeStruct((R, CW), jnp.float32))
        scratch += [
            pltpu.VMEM((CH, CW), jnp.float32),
            pltpu.VMEM((CH, CW), jnp.float32),
            pltpu.VMEM_SHARED((R, CW), jnp.float32),
        ]

    @functools.partial(pl.kernel, out_type=out_type, mesh=mesh,
                       scratch_types=scratch,
                       compiler_params=pltpu.CompilerParams(
                           use_tc_tiling_on_sc=False))
    def k(table_h, srcs_h, dst_h, zeros_h, zo16_h, *refs):
        if with_counts:
            (acc_out, cnt_out, src_v, dst_v, rows_v, zero_v, sem, acc_sh,
             ones_v, cntb_v, cnt_sh) = refs
        else:
            acc_out, src_v, dst_v, rows_v, zero_v, sem, acc_sh = refs
        cid = lax.axis_index("c")
        sid = lax.axis_index("s")

        pltpu.sync_copy(zeros_h, zero_v)
        for t in range(RPT // CH):
            pltpu.sync_copy(zero_v, acc_sh.at[pl.ds(sid * RPT + t * CH, CH)])
        if with_counts:
            pltpu.sync_copy(zo16_h.at[pl.ds(CH, CH)], ones_v)
            pltpu.sync_copy(zo16_h.at[pl.ds(0, CH)], cntb_v)
            for t in range(RPT // CH):
                pltpu.sync_copy(cntb_v, cnt_sh.at[pl.ds(sid * RPT + t * CH, CH)])
        plsc.subcore_barrier()

        base = sid * ET

        def chunk(j, carry):
            off = base + j * CH
            pltpu.sync_copy(srcs_h.at[cid, pl.ds(off, CH)], src_v)
            pltpu.sync_copy(dst_h.at[pl.ds(off, CH)], dst_v)
            gat = pltpu.async_copy(table_h.at[src_v], rows_v, sem)
            if with_counts:
                @pl.when(cid == 0)
                def _():
                    pltpu.sync_copy(ones_v, cnt_sh.at[dst_v], add=True)
            gat.wait()
            pltpu.sync_copy(rows_v, acc_sh.at[dst_v], add=True)
            return carry

        lax.fori_loop(0, ECHUNKS, chunk, 0)
        plsc.subcore_barrier()
        for t in range(RPT // CH):
            pltpu.sync_copy(acc_sh.at[pl.ds(sid * RPT + t * CH, CH)], rows_v)
            pltpu.sync_copy(rows_v, acc_out.at[cid, pl.ds(sid * RPT + t * CH, CH)])
        if with_counts:
            @pl.when(cid == 0)
            def _():
                for t in range(RPT // CH):
                    pltpu.sync_copy(cnt_sh.at[pl.ds(sid * RPT + t * CH, CH)], cntb_v)
                    pltpu.sync_copy(cntb_v, cnt_out.at[pl.ds(sid * RPT + t * CH, CH)])

    return k


_sc_agg_cnt = _make_sc_agg(True)
_sc_agg = _make_sc_agg(False)


BM = 1024


def _tc_linear(x, W, b):
    Do = W.shape[1]

    def body(x_r, w_r, b_r, o_r):
        o_r[...] = jnp.dot(x_r[...], w_r[...],
                           preferred_element_type=jnp.float32) + b_r[...]

    return pl.pallas_call(
        body,
        grid=(R // BM,),
        in_specs=[pl.BlockSpec((BM, D), lambda i: (i, 0)),
                  pl.BlockSpec((D, Do), lambda i: (0, 0)),
                  pl.BlockSpec((1, Do), lambda i: (0, 0))],
        out_specs=pl.BlockSpec((BM, Do), lambda i: (i, 0)),
        out_shape=jax.ShapeDtypeStruct((R, Do), jnp.float32),
    )(x, W, b)


def _tc_combine(a0, a1, c, pre, Wl):
    def body(a0r, a1r, cr, prer, wr, o_r):
        r = 1.0 / jnp.maximum(cr[...], 1.0)
        mean = jnp.concatenate([a0r[...], a1r[...]], axis=-1) * r
        o_r[...] = jnp.maximum(
            jnp.dot(mean, wr[...], preferred_element_type=jnp.float32)
            + prer[...], 0.0)

    return pl.pallas_call(
        body,
        grid=(R // BM,),
        in_specs=[pl.BlockSpec((BM, HD), lambda i: (i, 0)),
                  pl.BlockSpec((BM, HD), lambda i: (i, 0)),
                  pl.BlockSpec((BM, 1), lambda i: (i, 0)),
                  pl.BlockSpec((BM, D), lambda i: (i, 0)),
                  pl.BlockSpec((D, D), lambda i: (0, 0))],
        out_specs=pl.BlockSpec((BM, D), lambda i: (i, 0)),
        out_shape=jax.ShapeDtypeStruct((R, D), jnp.float32),
    )(a0, a1, c, pre, Wl)


def _tc_heads(a0, a1, c, pre, W2l, Wt, bt, Wi, bi, Wmh, Wmt, Wmi, bm):
    NT, NI, NM = Wt.shape[1], Wi.shape[1], Wmh.shape[1]

    def body(a0r, a1r, cr, prer, w2r, wtr, btr, wir, bir,
             wmhr, wmtr, wmir, bmr, tlr, ilr, imr):
        r = 1.0 / jnp.maximum(cr[...], 1.0)
        mean = jnp.concatenate([a0r[...], a1r[...]], axis=-1) * r
        h = jnp.maximum(
            jnp.dot(mean, w2r[...], preferred_element_type=jnp.float32)
            + prer[...], 0.0)
        tl = jnp.dot(h, wtr[...], preferred_element_type=jnp.float32) + btr[...]
        il = jnp.dot(h, wir[...], preferred_element_type=jnp.float32) + bir[...]
        tp = 1.0 / (1.0 + jnp.exp(-tl))
        m = jnp.max(il, axis=-1, keepdims=True)
        e = jnp.exp(il - m)
        ip = e / jnp.sum(e, axis=-1, keepdims=True)
        im = (jnp.dot(h, wmhr[...], preferred_element_type=jnp.float32)
              + jnp.dot(tp, wmtr[...], preferred_element_type=jnp.float32)
              + jnp.dot(ip, wmir[...], preferred_element_type=jnp.float32)
              + bmr[...])
        tlr[...] = tl
        ilr[...] = il
        imr[...] = im

    full = lambda shape: pl.BlockSpec(shape, lambda i: (0, 0))
    row = lambda w: pl.BlockSpec((BM, w), lambda i: (i, 0))
    return pl.pallas_call(
        body,
        grid=(R // BM,),
        in_specs=[row(HD), row(HD), row(1), row(D),
                  full((D, D)),
                  full((D, NT)), full((1, NT)),
                  full((D, NI)), full((1, NI)),
                  full((D, NM)), full((NT, NM)), full((NI, NM)), full((1, NM))],
        out_specs=[row(NT), row(NI), row(NM)],
        out_shape=[jax.ShapeDtypeStruct((R, NT), jnp.float32),
                   jax.ShapeDtypeStruct((R, NI), jnp.float32),
                   jax.ShapeDtypeStruct((R, NM), jnp.float32)],
    )(a0, a1, c, pre, W2l, Wt, bt, Wi, bi, Wmh, Wmt, Wmi, bm)


def kernel(x, edge_index, W1l, W1r, b1, W2l, W2r, b2, Wt, bt, Wi, bi, Wm, bm):
    x_pad = jnp.pad(x, ((0, R - N), (0, 0)))
    src = edge_index[0]
    dst = edge_index[1]
    pad_n = EPAD - E
    src_p = jnp.concatenate([src, jnp.zeros((pad_n,), jnp.int32)])
    dst_p = jnp.concatenate(
        [dst, N + (jnp.arange(pad_n, dtype=jnp.int32) % (R - N))])
    srcs = jnp.stack([2 * src_p, 2 * src_p + 1])
    zeros = jnp.zeros((CH, HD), jnp.float32)
    zo16 = jnp.concatenate([jnp.zeros((CH, CW), jnp.float32),
                            jnp.ones((CH, CW), jnp.float32)])

    pre1 = _tc_linear(x_pad, W1r, b1.reshape(1, D))
    a1, cnt = _sc_agg_cnt(x_pad.reshape(2 * R, HD), srcs, dst_p, zeros, zo16)
    c = cnt[:, 0].reshape(R, 1)
    h1 = _tc_combine(a1[0], a1[1], c, pre1, W1l)

    pre2 = _tc_linear(h1, W2r, b2.reshape(1, D))
    a2, = _sc_agg(h1.reshape(2 * R, HD), srcs, dst_p, zeros, zo16)
    tl, il, im = _tc_heads(
        a2[0], a2[1], c, pre2, W2l,
        Wt, bt.reshape(1, -1), Wi, bi.reshape(1, -1),
        Wm[:D], Wm[D:D + 32], Wm[D + 32:], bm.reshape(1, -1))
    return (tl[:N], il[:N], im[:N])
